# Initial kernel scaffold; baseline (speedup 1.0000x reference)
import jax
import jax.numpy as jnp
from jax.experimental import pallas as pl


def kernel(code, embedding):
    raise NotImplementedError("write your pallas kernel here")


# speedup vs baseline: 4.1146x; 4.1146x over previous
import functools

import jax
import jax.numpy as jnp
from jax import lax
from jax.experimental import pallas as pl
from jax.experimental.pallas import tpu as pltpu
from jax.experimental.pallas import tpu_sc as plsc

VOCAB = 100000
D = 64
BATCH = 4096
SEQ = 200
B_TOTAL = BATCH * SEQ

NC = 2
NS = 16
NW = NC * NS

CHUNK = 128
PER_W = B_TOTAL // NW
NCHUNK = PER_W // CHUNK


def _make_sc_gather():
    mesh = plsc.VectorSubcoreMesh(
        core_axis_name="c", subcore_axis_name="s", num_cores=NC, num_subcores=NS
    )

    @functools.partial(
        pl.kernel,
        mesh=mesh,
        out_type=jax.ShapeDtypeStruct((B_TOTAL, D), jnp.float32),
        scratch_types=[
            pltpu.VMEM((NCHUNK, CHUNK), jnp.int32),
            pltpu.VMEM((2, CHUNK, D), jnp.float32),
            pltpu.SemaphoreType.DMA,
            pltpu.SemaphoreType.DMA,
        ],
        compiler_params=pltpu.CompilerParams(use_tc_tiling_on_sc=False),
    )
    def k(code_hbm, table_hbm, out_hbm, idx_v, rows_v, gsem0, gsem1):
        wid = lax.axis_index("s") * NC + lax.axis_index("c")
        base = wid * PER_W
        pltpu.sync_copy(code_hbm.at[wid], idx_v)

        gsem = (gsem0, gsem1)

        def start_gather(j, b):
            pltpu.async_copy(table_hbm.at[idx_v.at[j]], rows_v.at[b], gsem[b])

        def wait_gather(j, b):
            pltpu.make_async_copy(
                table_hbm.at[idx_v.at[j]], rows_v.at[b], gsem[b]
            ).wait()

        def write_out(j, b):
            pltpu.sync_copy(
                rows_v.at[b], out_hbm.at[pl.ds(base + j * CHUNK, CHUNK)]
            )

        start_gather(0, 0)
        start_gather(1, 1)
        wait_gather(0, 0)
        write_out(0, 0)

        def pair(i, carry):
            for t in range(2):
                j = 1 + 2 * i + t
                b = (1 + t) % 2
                nb = t % 2
                start_gather(j + 1, nb)
                wait_gather(j, b)
                write_out(j, b)
            return carry

        lax.fori_loop(0, (NCHUNK - 2) // 2, pair, 0)

        wait_gather(NCHUNK - 1, 1)
        write_out(NCHUNK - 1, 1)

    return k


_sc_gather = _make_sc_gather()


def kernel(code, embedding):
    code3 = code.reshape(NW, NCHUNK, CHUNK).astype(jnp.int32)
    out = _sc_gather(code3, embedding)
    return out.reshape(BATCH, SEQ, D)


# speedup vs baseline: 4.2635x; 1.0362x over previous
import functools

import jax
import jax.numpy as jnp
from jax import lax
from jax.experimental import pallas as pl
from jax.experimental.pallas import tpu as pltpu
from jax.experimental.pallas import tpu_sc as plsc

VOCAB = 100000
D = 64
BATCH = 4096
SEQ = 200
B_TOTAL = BATCH * SEQ

NC = 2
NS = 16
NW = NC * NS

CHUNK = 128
PER_W = B_TOTAL // NW
NCHUNK = PER_W // CHUNK
NBUF = 4


def _make_sc_gather():
    mesh = plsc.VectorSubcoreMesh(
        core_axis_name="c", subcore_axis_name="s", num_cores=NC, num_subcores=NS
    )

    @functools.partial(
        pl.kernel,
        mesh=mesh,
        out_type=jax.ShapeDtypeStruct((B_TOTAL, D), jnp.float32),
        scratch_types=[
            pltpu.VMEM((NCHUNK, CHUNK), jnp.int32),
            pltpu.VMEM((NBUF, CHUNK, D), jnp.float32),
        ] + [pltpu.SemaphoreType.DMA] * NBUF,
        compiler_params=pltpu.CompilerParams(use_tc_tiling_on_sc=False),
    )
    def k(code_hbm, table_hbm, out_hbm, idx_v, rows_v, *gsem):
        wid = lax.axis_index("s") * NC + lax.axis_index("c")
        base = wid * PER_W
        pltpu.sync_copy(code_hbm.at[wid], idx_v)

        def start_gather(j, b):
            pltpu.async_copy(table_hbm.at[idx_v.at[j]], rows_v.at[b], gsem[b])

        def wait_gather(j, b):
            pltpu.make_async_copy(
                table_hbm.at[idx_v.at[j]], rows_v.at[b], gsem[b]
            ).wait()

        def write_out(j, b):
            pltpu.sync_copy(
                rows_v.at[b], out_hbm.at[pl.ds(base + j * CHUNK, CHUNK)]
            )

        for t in range(NBUF - 1):
            start_gather(t, t)

        M = (NCHUNK - (NBUF - 1)) // NBUF

        def block(i, carry):
            for t in range(NBUF):
                j = i * NBUF + t
                start_gather(j + NBUF - 1, (t + NBUF - 1) % NBUF)
                wait_gather(j, t)
                write_out(j, t)
            return carry

        lax.fori_loop(0, M, block, 0)

        for j in range(M * NBUF, NCHUNK):
            b = j % NBUF
            if j + NBUF - 1 < NCHUNK:
                start_gather(j + NBUF - 1, (j + NBUF - 1) % NBUF)
            wait_gather(j, b)
            write_out(j, b)

    return k


_sc_gather = _make_sc_gather()


def kernel(code, embedding):
    code3 = code.reshape(NW, NCHUNK, CHUNK).astype(jnp.int32)
    out = _sc_gather(code3, embedding)
    return out.reshape(BATCH, SEQ, D)
